# Initial kernel scaffold; baseline (speedup 1.0000x reference)
import functools

import jax
import jax.numpy as jnp
from jax import lax
from jax.experimental import pallas as pl
from jax.experimental.pallas import tpu as pltpu
from jax.experimental.pallas import tpu_sc as plsc

N, NSUB, CIN, COUT = 10000, 2500, 512, 256
NP = 10240
NSUBP = 2560
QBLK = 512
NQB = NP // QBLK
DBLK = 1024
NDB = NP // DBLK

_F32 = jnp.float32
_HI = lax.Precision.HIGHEST


def _mlp_sub_body(xs_ref, w_ref, b_ref, g_ref, be_ref, h_ref):
    y = jnp.dot(xs_ref[...], w_ref[...], preferred_element_type=_F32,
                precision=_HI) + b_ref[...]
    rid = lax.broadcasted_iota(jnp.int32, (NSUBP, 1), 0)
    m = rid < NSUB
    ym = jnp.where(m, y, 0.0)
    mean = jnp.sum(ym, axis=0, keepdims=True) / NSUB
    dev = jnp.where(m, y - mean, 0.0)
    var = jnp.sum(dev * dev, axis=0, keepdims=True) / NSUB
    hn = (y - mean) / jnp.sqrt(var + 1e-5)
    h = jnp.maximum(hn * g_ref[...] + be_ref[...], 0.0)
    h_ref[...] = jnp.where(m, h, 0.0)


def _dense_body(xb_ref, w_ref, b_ref, g_ref, be_ref, y_ref, stats_ref, acc_ref):
    j = pl.program_id(0)
    y = jnp.dot(xb_ref[...], w_ref[...], preferred_element_type=_F32,
                precision=_HI) + b_ref[...]
    y_ref[...] = y
    rid = lax.broadcasted_iota(jnp.int32, (DBLK, 1), 0)
    m = rid < (N - j * DBLK)
    ym = jnp.where(m, y, 0.0)

    @pl.when(j == 0)
    def _():
        acc_ref[...] = jnp.zeros_like(acc_ref)

    acc_ref[0:1, :] += jnp.sum(ym, axis=0, keepdims=True)
    acc_ref[1:2, :] += jnp.sum(ym * ym, axis=0, keepdims=True)

    @pl.when(j == NDB - 1)
    def _():
        mean = acc_ref[0:1, :] / N
        var = acc_ref[1:2, :] / N - mean * mean
        scale = g_ref[...] / jnp.sqrt(var + 1e-5)
        shift = be_ref[...] - mean * scale
        stats_ref[0:1, :] = scale
        stats_ref[1:2, :] = shift


def _knn_body(pq_ref, ps_ref, idx_ref, wn_ref):
    qx = pq_ref[:, 0:1]
    qy = pq_ref[:, 1:2]
    qz = pq_ref[:, 2:3]
    sx = ps_ref[0:1, :]
    sy = ps_ref[1:2, :]
    sz = ps_ref[2:3, :]
    d2 = ((qx * qx + qy * qy + qz * qz)
          + (sx * sx + sy * sy + sz * sz)
          - 2.0 * (qx * sx + qy * sy + qz * sz))
    ids = lax.broadcasted_iota(jnp.int32, (QBLK, NSUBP), 1)
    d = d2
    ams, ws = [], []
    for _ in range(3):
        mval = jnp.min(d, axis=1, keepdims=True)
        am = jnp.min(jnp.where(d == mval, ids, jnp.int32(2**30)),
                     axis=1, keepdims=True)
        ams.append(am)
        ws.append(1.0 / (jnp.maximum(mval, 0.0) + 1e-16))
        d = jnp.where(ids == am, jnp.float32(3e38), d)
    wsum = ws[0] + ws[1] + ws[2]
    zi = jnp.zeros((QBLK, 5), jnp.int32)
    zf = jnp.zeros((QBLK, 5), _F32)
    idx_ref[...] = jnp.concatenate(ams + [zi], axis=1)
    wn_ref[...] = jnp.concatenate([w / wsum for w in ws] + [zf], axis=1)


_NC, _NS = 2, 16
_NW = _NC * _NS
_RPW = NP // _NW
_CH = 64
_NCH = _RPW // _CH


def _sc_interp_body(i0_hbm, i1_hbm, i2_hbm, w0_hbm, w1_hbm, w2_hbm,
                    h_hbm, y_hbm, sc_hbm, sh_hbm, out_hbm,
                    i0_v, i1_v, i2_v, w0_v, w1_v, w2_v,
                    r0_v, r1_v, r2_v, y_v, out_v, sc_v, sh_v, sem):
    wid = lax.axis_index("s") * _NC + lax.axis_index("c")
    pltpu.sync_copy(sc_hbm, sc_v)
    pltpu.sync_copy(sh_hbm, sh_v)
    base0 = wid * _RPW
    for chunk in range(_NCH):
        base = base0 + chunk * _CH
        sl = pl.ds(base, _CH)
        pltpu.sync_copy(i0_hbm.at[sl], i0_v)
        pltpu.sync_copy(i1_hbm.at[sl], i1_v)
        pltpu.sync_copy(i2_hbm.at[sl], i2_v)
        pltpu.sync_copy(w0_hbm.at[sl], w0_v)
        pltpu.sync_copy(w1_hbm.at[sl], w1_v)
        pltpu.sync_copy(w2_hbm.at[sl], w2_v)
        c0 = pltpu.async_copy(h_hbm.at[i0_v], r0_v, sem)
        c1 = pltpu.async_copy(h_hbm.at[i1_v], r1_v, sem)
        c2 = pltpu.async_copy(h_hbm.at[i2_v], r2_v, sem)
        pltpu.sync_copy(y_hbm.at[sl], y_v)
        c0.wait()
        c1.wait()
        c2.wait()

        def qbody(q, carry):
            qi = jnp.full((16,), q, jnp.int32)
            w0 = plsc.load_gather(w0_v, [qi])
            w1 = plsc.load_gather(w1_v, [qi])
            w2 = plsc.load_gather(w2_v, [qi])
            for c in range(COUT // 16):
                cs = pl.ds(c * 16, 16)
                dn = jnp.maximum(y_v[q, cs] * sc_v[cs] + sh_v[cs], 0.0)
                out_v[q, cs] = (dn + w0 * r0_v[q, cs] + w1 * r1_v[q, cs]
                                + w2 * r2_v[q, cs])
            return carry

        lax.fori_loop(0, _CH, qbody, 0)
        pltpu.sync_copy(out_v, out_hbm.at[sl])


def _sc_interp(idx0, idx1, idx2, wn0, wn1, wn2, h, ybuf, scale, shift):
    mesh = plsc.VectorSubcoreMesh(core_axis_name="c", subcore_axis_name="s")
    kfn = pl.kernel(
        _sc_interp_body,
        mesh=mesh,
        out_type=jax.ShapeDtypeStruct((NP, COUT), _F32),
        scratch_types=[
            pltpu.VMEM((_CH,), jnp.int32),
            pltpu.VMEM((_CH,), jnp.int32),
            pltpu.VMEM((_CH,), jnp.int32),
            pltpu.VMEM((_CH,), _F32),
            pltpu.VMEM((_CH,), _F32),
            pltpu.VMEM((_CH,), _F32),
            pltpu.VMEM((_CH, COUT), _F32),
            pltpu.VMEM((_CH, COUT), _F32),
            pltpu.VMEM((_CH, COUT), _F32),
            pltpu.VMEM((_CH, COUT), _F32),
            pltpu.VMEM((_CH, COUT), _F32),
            pltpu.VMEM((COUT,), _F32),
            pltpu.VMEM((COUT,), _F32),
            pltpu.SemaphoreType.DMA,
        ],
    )
    return kfn(idx0, idx1, idx2, wn0, wn1, wn2, h, ybuf, scale, shift)


@jax.jit
def kernel(x, x_sub, pos, pos_sub, W_sub, b_sub, g_sub, be_sub, W, b, g, be):
    xs_pad = jnp.zeros((NSUBP, CIN), _F32).at[:NSUB].set(x_sub)
    x_pad = jnp.zeros((NP, COUT), _F32).at[:N].set(x)
    posq = jnp.zeros((NP, 8), _F32).at[:N, :3].set(pos)
    poss = jnp.full((8, NSUBP), 1e3, _F32).at[:3, :NSUB].set(pos_sub.T)

    h = pl.pallas_call(
        _mlp_sub_body,
        out_shape=jax.ShapeDtypeStruct((NSUBP, COUT), _F32),
    )(xs_pad, W_sub, b_sub, g_sub, be_sub)

    ybuf, stats = pl.pallas_call(
        _dense_body,
        grid=(NDB,),
        in_specs=[
            pl.BlockSpec((DBLK, COUT), lambda j: (j, 0)),
            pl.BlockSpec((COUT, COUT), lambda j: (0, 0)),
            pl.BlockSpec((1, COUT), lambda j: (0, 0)),
            pl.BlockSpec((1, COUT), lambda j: (0, 0)),
            pl.BlockSpec((1, COUT), lambda j: (0, 0)),
        ],
        out_specs=[
            pl.BlockSpec((DBLK, COUT), lambda j: (j, 0)),
            pl.BlockSpec((8, COUT), lambda j: (0, 0)),
        ],
        out_shape=[
            jax.ShapeDtypeStruct((NP, COUT), _F32),
            jax.ShapeDtypeStruct((8, COUT), _F32),
        ],
        scratch_shapes=[pltpu.VMEM((8, COUT), _F32)],
    )(x_pad, W, b.reshape(1, COUT), g.reshape(1, COUT), be.reshape(1, COUT))

    idxs, wns = pl.pallas_call(
        _knn_body,
        grid=(NQB,),
        in_specs=[
            pl.BlockSpec((QBLK, 8), lambda j: (j, 0)),
            pl.BlockSpec((8, NSUBP), lambda j: (0, 0)),
        ],
        out_specs=[
            pl.BlockSpec((QBLK, 8), lambda j: (j, 0)),
            pl.BlockSpec((QBLK, 8), lambda j: (j, 0)),
        ],
        out_shape=[
            jax.ShapeDtypeStruct((NP, 8), jnp.int32),
            jax.ShapeDtypeStruct((NP, 8), _F32),
        ],
    )(posq, poss)

    out = _sc_interp(
        jnp.ascontiguousarray(idxs[:, 0]), jnp.ascontiguousarray(idxs[:, 1]),
        jnp.ascontiguousarray(idxs[:, 2]), jnp.ascontiguousarray(wns[:, 0]),
        jnp.ascontiguousarray(wns[:, 1]), jnp.ascontiguousarray(wns[:, 2]),
        h, ybuf, jnp.ascontiguousarray(stats[0]), jnp.ascontiguousarray(stats[1]))
    return out[:N]


# speedup vs baseline: 2.4425x; 2.4425x over previous
import functools

import jax
import jax.numpy as jnp
from jax import lax
from jax.experimental import pallas as pl
from jax.experimental.pallas import tpu as pltpu
from jax.experimental.pallas import tpu_sc as plsc

N, NSUB, CIN, COUT = 10000, 2500, 512, 256
NP = 10240
NSUBP = 2560
QBLK = 512
NQB = NP // QBLK
DBLK = 1024
NDB = NP // DBLK

_F32 = jnp.float32
_HI = lax.Precision.HIGHEST


def _mlp_sub_body(xs_ref, w_ref, b_ref, g_ref, be_ref, h_ref):
    y = jnp.dot(xs_ref[...], w_ref[...],
                preferred_element_type=_F32) + b_ref[...]
    rid = lax.broadcasted_iota(jnp.int32, (NSUBP, 1), 0)
    m = rid < NSUB
    ym = jnp.where(m, y, 0.0)
    mean = jnp.sum(ym, axis=0, keepdims=True) / NSUB
    dev = jnp.where(m, y - mean, 0.0)
    var = jnp.sum(dev * dev, axis=0, keepdims=True) / NSUB
    hn = (y - mean) / jnp.sqrt(var + 1e-5)
    h = jnp.maximum(hn * g_ref[...] + be_ref[...], 0.0)
    h_ref[...] = jnp.where(m, h, 0.0)


def _dense_body(xb_ref, w_ref, b_ref, g_ref, be_ref, y_ref, stats_ref, acc_ref):
    j = pl.program_id(0)
    y = jnp.dot(xb_ref[...], w_ref[...],
                preferred_element_type=_F32) + b_ref[...]
    y_ref[...] = y
    rid = lax.broadcasted_iota(jnp.int32, (DBLK, 1), 0)
    m = rid < (N - j * DBLK)
    ym = jnp.where(m, y, 0.0)

    @pl.when(j == 0)
    def _():
        acc_ref[...] = jnp.zeros_like(acc_ref)

    acc_ref[0:1, :] += jnp.sum(ym, axis=0, keepdims=True)
    acc_ref[1:2, :] += jnp.sum(ym * ym, axis=0, keepdims=True)

    @pl.when(j == NDB - 1)
    def _():
        mean = acc_ref[0:1, :] / N
        var = acc_ref[1:2, :] / N - mean * mean
        scale = g_ref[...] / jnp.sqrt(var + 1e-5)
        shift = be_ref[...] - mean * scale
        stats_ref[0:1, :] = scale
        stats_ref[1:2, :] = shift


def _knn_body(pq_ref, ps_ref, idx_ref, wn_ref):
    qx = pq_ref[:, 0:1]
    qy = pq_ref[:, 1:2]
    qz = pq_ref[:, 2:3]
    sx = ps_ref[0:1, :]
    sy = ps_ref[1:2, :]
    sz = ps_ref[2:3, :]
    def _bf(v):
        return v.astype(jnp.bfloat16).astype(_F32)
    dot = _bf(qx) * _bf(sx) + _bf(qy) * _bf(sy) + _bf(qz) * _bf(sz)
    d2 = ((qx * qx + qy * qy + qz * qz)
          + (sx * sx + sy * sy + sz * sz)
          - 2.0 * dot)
    ids = lax.broadcasted_iota(jnp.int32, (QBLK, NSUBP), 1)
    d = d2
    ams, ws = [], []
    for _ in range(3):
        mval = jnp.min(d, axis=1, keepdims=True)
        am = jnp.min(jnp.where(d == mval, ids, jnp.int32(2**30)),
                     axis=1, keepdims=True)
        ams.append(am)
        ws.append(1.0 / (jnp.maximum(mval, 0.0) + 1e-16))
        d = jnp.where(ids == am, jnp.float32(3e38), d)
    wsum = ws[0] + ws[1] + ws[2]
    zi = jnp.zeros((QBLK, 5), jnp.int32)
    idx_ref[...] = jnp.concatenate(ams + [zi], axis=1)
    wn_ref[...] = jnp.concatenate(
        [jnp.broadcast_to(w / wsum, (QBLK, 16)) for w in ws], axis=1)


_NC, _NS = 2, 16
_NW = _NC * _NS
_RPW = NP // _NW
_CH = 64
_NCH = _RPW // _CH


def _sc_interp_body(i0_hbm, i1_hbm, i2_hbm, wb_hbm,
                    h_hbm, y_hbm, sc_hbm, sh_hbm, out_hbm,
                    i0_v, i1_v, i2_v, wb_v,
                    r0_v, r1_v, r2_v, y_v, out_v, sc_v, sh_v, sem):
    wid = lax.axis_index("s") * _NC + lax.axis_index("c")
    pltpu.sync_copy(sc_hbm, sc_v)
    pltpu.sync_copy(sh_hbm, sh_v)
    base0 = wid * _RPW
    for chunk in range(_NCH):
        base = base0 + chunk * _CH
        sl = pl.ds(base, _CH)
        pltpu.sync_copy(i0_hbm.at[sl], i0_v)
        pltpu.sync_copy(i1_hbm.at[sl], i1_v)
        pltpu.sync_copy(i2_hbm.at[sl], i2_v)
        pltpu.sync_copy(wb_hbm.at[sl], wb_v)
        c0 = pltpu.async_copy(h_hbm.at[i0_v], r0_v, sem)
        c1 = pltpu.async_copy(h_hbm.at[i1_v], r1_v, sem)
        c2 = pltpu.async_copy(h_hbm.at[i2_v], r2_v, sem)
        pltpu.sync_copy(y_hbm.at[sl], y_v)
        c0.wait()
        c1.wait()
        c2.wait()

        def qbody(q, carry):
            w0 = wb_v[q, pl.ds(0, 16)]
            w1 = wb_v[q, pl.ds(16, 16)]
            w2 = wb_v[q, pl.ds(32, 16)]
            for c in range(COUT // 16):
                cs = pl.ds(c * 16, 16)
                dn = jnp.maximum(y_v[q, cs] * sc_v[cs] + sh_v[cs], 0.0)
                out_v[q, cs] = (dn + w0 * r0_v[q, cs] + w1 * r1_v[q, cs]
                                + w2 * r2_v[q, cs])
            return carry

        lax.fori_loop(0, _CH, qbody, 0)
        pltpu.sync_copy(out_v, out_hbm.at[sl])


def _sc_interp(idx0, idx1, idx2, wnb, h, ybuf, scale, shift):
    mesh = plsc.VectorSubcoreMesh(core_axis_name="c", subcore_axis_name="s")
    kfn = pl.kernel(
        _sc_interp_body,
        mesh=mesh,
        out_type=jax.ShapeDtypeStruct((NP, COUT), _F32),
        scratch_types=[
            pltpu.VMEM((_CH,), jnp.int32),
            pltpu.VMEM((_CH,), jnp.int32),
            pltpu.VMEM((_CH,), jnp.int32),
            pltpu.VMEM((_CH, 48), _F32),
            pltpu.VMEM((_CH, COUT), _F32),
            pltpu.VMEM((_CH, COUT), _F32),
            pltpu.VMEM((_CH, COUT), _F32),
            pltpu.VMEM((_CH, COUT), _F32),
            pltpu.VMEM((_CH, COUT), _F32),
            pltpu.VMEM((COUT,), _F32),
            pltpu.VMEM((COUT,), _F32),
            pltpu.SemaphoreType.DMA,
        ],
    )
    return kfn(idx0, idx1, idx2, wnb, h, ybuf, scale, shift)


@jax.jit
def kernel(x, x_sub, pos, pos_sub, W_sub, b_sub, g_sub, be_sub, W, b, g, be):
    xs_pad = jnp.zeros((NSUBP, CIN), _F32).at[:NSUB].set(x_sub)
    x_pad = jnp.zeros((NP, COUT), _F32).at[:N].set(x)
    posq = jnp.zeros((NP, 8), _F32).at[:N, :3].set(pos)
    poss = jnp.full((8, NSUBP), 1e3, _F32).at[:3, :NSUB].set(pos_sub.T)

    h = pl.pallas_call(
        _mlp_sub_body,
        out_shape=jax.ShapeDtypeStruct((NSUBP, COUT), _F32),
    )(xs_pad, W_sub, b_sub, g_sub, be_sub)

    ybuf, stats = pl.pallas_call(
        _dense_body,
        grid=(NDB,),
        in_specs=[
            pl.BlockSpec((DBLK, COUT), lambda j: (j, 0)),
            pl.BlockSpec((COUT, COUT), lambda j: (0, 0)),
            pl.BlockSpec((1, COUT), lambda j: (0, 0)),
            pl.BlockSpec((1, COUT), lambda j: (0, 0)),
            pl.BlockSpec((1, COUT), lambda j: (0, 0)),
        ],
        out_specs=[
            pl.BlockSpec((DBLK, COUT), lambda j: (j, 0)),
            pl.BlockSpec((8, COUT), lambda j: (0, 0)),
        ],
        out_shape=[
            jax.ShapeDtypeStruct((NP, COUT), _F32),
            jax.ShapeDtypeStruct((8, COUT), _F32),
        ],
        scratch_shapes=[pltpu.VMEM((8, COUT), _F32)],
    )(x_pad, W, b.reshape(1, COUT), g.reshape(1, COUT), be.reshape(1, COUT))

    idxs, wns = pl.pallas_call(
        _knn_body,
        grid=(NQB,),
        in_specs=[
            pl.BlockSpec((QBLK, 8), lambda j: (j, 0)),
            pl.BlockSpec((8, NSUBP), lambda j: (0, 0)),
        ],
        out_specs=[
            pl.BlockSpec((QBLK, 8), lambda j: (j, 0)),
            pl.BlockSpec((QBLK, 48), lambda j: (j, 0)),
        ],
        out_shape=[
            jax.ShapeDtypeStruct((NP, 8), jnp.int32),
            jax.ShapeDtypeStruct((NP, 48), _F32),
        ],
    )(posq, poss)

    out = _sc_interp(
        idxs[:, 0], idxs[:, 1], idxs[:, 2], wns,
        h, ybuf, stats[0], stats[1])
    return out[:N]


# speedup vs baseline: 3.1214x; 1.2779x over previous
import functools

import jax
import jax.numpy as jnp
from jax import lax
from jax.experimental import pallas as pl
from jax.experimental.pallas import tpu as pltpu
from jax.experimental.pallas import tpu_sc as plsc

N, NSUB, CIN, COUT = 10000, 2500, 512, 256
NP = 10240
NSUBP = 2560
QBLK = 512
NQB = NP // QBLK
DBLK = 1024
NDB = NP // DBLK

_F32 = jnp.float32
_HI = lax.Precision.HIGHEST


def _mlp_sub_body(xs_ref, w_ref, b_ref, g_ref, be_ref, h_ref):
    y = jnp.dot(xs_ref[...], w_ref[...],
                preferred_element_type=_F32) + b_ref[...]
    mean = jnp.sum(y, axis=0, keepdims=True) / NSUB
    dev = y - mean
    var = jnp.sum(dev * dev, axis=0, keepdims=True) / NSUB
    hn = dev / jnp.sqrt(var + 1e-5)
    h_ref[...] = jnp.maximum(hn * g_ref[...] + be_ref[...], 0.0)


def _dense_body(xb_ref, w_ref, b_ref, g_ref, be_ref, y_ref, stats_ref, acc_ref):
    j = pl.program_id(0)
    y = jnp.dot(xb_ref[...], w_ref[...],
                preferred_element_type=_F32) + b_ref[...]
    y_ref[...] = y
    rid = lax.broadcasted_iota(jnp.int32, (DBLK, 1), 0)
    m = rid < (N - j * DBLK)
    ym = jnp.where(m, y, 0.0)

    @pl.when(j == 0)
    def _():
        acc_ref[...] = jnp.zeros_like(acc_ref)

    acc_ref[0:1, :] += jnp.sum(ym, axis=0, keepdims=True)
    acc_ref[1:2, :] += jnp.sum(ym * ym, axis=0, keepdims=True)

    @pl.when(j == NDB - 1)
    def _():
        mean = acc_ref[0:1, :] / N
        var = acc_ref[1:2, :] / N - mean * mean
        scale = g_ref[...] / jnp.sqrt(var + 1e-5)
        shift = be_ref[...] - mean * scale
        stats_ref[0:1, :] = scale
        stats_ref[1:2, :] = shift


def _knn_body(pq_ref, ps_ref, idx_ref, wn_ref):
    qx = pq_ref[:, 0:1]
    qy = pq_ref[:, 1:2]
    qz = pq_ref[:, 2:3]
    sx = ps_ref[0:1, :]
    sy = ps_ref[1:2, :]
    sz = ps_ref[2:3, :]
    def _bf(v):
        return v.astype(jnp.bfloat16).astype(_F32)
    dot = _bf(qx) * _bf(sx) + _bf(qy) * _bf(sy) + _bf(qz) * _bf(sz)
    d2 = ((qx * qx + qy * qy + qz * qz)
          + (sx * sx + sy * sy + sz * sz)
          - 2.0 * dot)
    ids = lax.broadcasted_iota(jnp.int32, (QBLK, NSUBP), 1)
    d = d2
    ams, ws = [], []
    for _ in range(3):
        mval = jnp.min(d, axis=1, keepdims=True)
        am = jnp.min(jnp.where(d == mval, ids, jnp.int32(2**30)),
                     axis=1, keepdims=True)
        ams.append(am)
        ws.append(1.0 / (jnp.maximum(mval, 0.0) + 1e-16))
        d = jnp.where(ids == am, jnp.float32(3e38), d)
    wsum = ws[0] + ws[1] + ws[2]
    zi = jnp.zeros((QBLK, 5), jnp.int32)
    idx_ref[...] = jnp.concatenate(ams + [zi], axis=1)
    wn_ref[...] = jnp.concatenate(
        [jnp.broadcast_to(w / wsum, (QBLK, 16)) for w in ws], axis=1)


_NC, _NS = 2, 16
_NW = _NC * _NS
_RPW = NP // _NW
_CH = 64
_NCH = _RPW // _CH


def _sc_interp_body(i0_hbm, i1_hbm, i2_hbm, wb_hbm, h_hbm, out_hbm,
                    i0_v, i1_v, i2_v, wb_v,
                    r0_v, r1_v, r2_v, out_v, sem):
    wid = lax.axis_index("s") * _NC + lax.axis_index("c")
    base0 = wid * _RPW
    for chunk in range(_NCH):
        base = base0 + chunk * _CH
        sl = pl.ds(base, _CH)
        pltpu.sync_copy(i0_hbm.at[sl], i0_v)
        pltpu.sync_copy(i1_hbm.at[sl], i1_v)
        pltpu.sync_copy(i2_hbm.at[sl], i2_v)
        c0 = pltpu.async_copy(h_hbm.at[i0_v], r0_v, sem)
        c1 = pltpu.async_copy(h_hbm.at[i1_v], r1_v, sem)
        c2 = pltpu.async_copy(h_hbm.at[i2_v], r2_v, sem)
        pltpu.sync_copy(wb_hbm.at[sl], wb_v)
        c0.wait()
        c1.wait()
        c2.wait()

        def qbody(q, carry):
            w0 = wb_v[q, pl.ds(0, 16)]
            w1 = wb_v[q, pl.ds(16, 16)]
            w2 = wb_v[q, pl.ds(32, 16)]
            for c in range(COUT // 16):
                cs = pl.ds(c * 16, 16)
                out_v[q, cs] = (w0 * r0_v[q, cs] + w1 * r1_v[q, cs]
                                + w2 * r2_v[q, cs])
            return carry

        lax.fori_loop(0, _CH, qbody, 0)
        pltpu.sync_copy(out_v, out_hbm.at[sl])


def _sc_interp(idx0, idx1, idx2, wnb, h):
    mesh = plsc.VectorSubcoreMesh(core_axis_name="c", subcore_axis_name="s")
    kfn = pl.kernel(
        _sc_interp_body,
        mesh=mesh,
        out_type=jax.ShapeDtypeStruct((NP, COUT), _F32),
        scratch_types=[
            pltpu.VMEM((_CH,), jnp.int32),
            pltpu.VMEM((_CH,), jnp.int32),
            pltpu.VMEM((_CH,), jnp.int32),
            pltpu.VMEM((_CH, 48), _F32),
            pltpu.VMEM((_CH, COUT), _F32),
            pltpu.VMEM((_CH, COUT), _F32),
            pltpu.VMEM((_CH, COUT), _F32),
            pltpu.VMEM((_CH, COUT), _F32),
            pltpu.SemaphoreType.DMA,
        ],
    )
    return kfn(idx0, idx1, idx2, wnb, h)


def _combine_body(y_ref, stats_ref, interp_ref, out_ref):
    scale = stats_ref[0:1, :]
    shift = stats_ref[1:2, :]
    dn = jnp.maximum(y_ref[...] * scale + shift, 0.0)
    out_ref[...] = dn + interp_ref[...]


@jax.jit
def kernel(x, x_sub, pos, pos_sub, W_sub, b_sub, g_sub, be_sub, W, b, g, be):
    posq = jnp.zeros((NP, 8), _F32).at[:N, :3].set(pos)
    poss = jnp.full((8, NSUBP), 1e3, _F32).at[:3, :NSUB].set(pos_sub.T)

    h = pl.pallas_call(
        _mlp_sub_body,
        out_shape=jax.ShapeDtypeStruct((NSUB, COUT), _F32),
    )(x_sub, W_sub, b_sub, g_sub, be_sub)

    ybuf, stats = pl.pallas_call(
        _dense_body,
        grid=(NDB,),
        in_specs=[
            pl.BlockSpec((DBLK, COUT), lambda j: (j, 0)),
            pl.BlockSpec((COUT, COUT), lambda j: (0, 0)),
            pl.BlockSpec((1, COUT), lambda j: (0, 0)),
            pl.BlockSpec((1, COUT), lambda j: (0, 0)),
            pl.BlockSpec((1, COUT), lambda j: (0, 0)),
        ],
        out_specs=[
            pl.BlockSpec((DBLK, COUT), lambda j: (j, 0)),
            pl.BlockSpec((8, COUT), lambda j: (0, 0)),
        ],
        out_shape=[
            jax.ShapeDtypeStruct((N, COUT), _F32),
            jax.ShapeDtypeStruct((8, COUT), _F32),
        ],
        scratch_shapes=[pltpu.VMEM((8, COUT), _F32)],
    )(x, W, b.reshape(1, COUT), g.reshape(1, COUT), be.reshape(1, COUT))

    idxs, wns = pl.pallas_call(
        _knn_body,
        grid=(NQB,),
        in_specs=[
            pl.BlockSpec((QBLK, 8), lambda j: (j, 0)),
            pl.BlockSpec((8, NSUBP), lambda j: (0, 0)),
        ],
        out_specs=[
            pl.BlockSpec((QBLK, 8), lambda j: (j, 0)),
            pl.BlockSpec((QBLK, 48), lambda j: (j, 0)),
        ],
        out_shape=[
            jax.ShapeDtypeStruct((NP, 8), jnp.int32),
            jax.ShapeDtypeStruct((NP, 48), _F32),
        ],
    )(posq, poss)

    interp = _sc_interp(idxs[:, 0], idxs[:, 1], idxs[:, 2], wns, h)

    out = pl.pallas_call(
        _combine_body,
        grid=(NDB,),
        in_specs=[
            pl.BlockSpec((DBLK, COUT), lambda j: (j, 0)),
            pl.BlockSpec((8, COUT), lambda j: (0, 0)),
            pl.BlockSpec((DBLK, COUT), lambda j: (j, 0)),
        ],
        out_specs=pl.BlockSpec((DBLK, COUT), lambda j: (j, 0)),
        out_shape=jax.ShapeDtypeStruct((N, COUT), _F32),
    )(ybuf, stats, interp)
    return out


# speedup vs baseline: 3.2604x; 1.0445x over previous
import functools

import jax
import jax.numpy as jnp
from jax import lax
from jax.experimental import pallas as pl
from jax.experimental.pallas import tpu as pltpu
from jax.experimental.pallas import tpu_sc as plsc

N, NSUB, CIN, COUT = 10000, 2500, 512, 256
NP = 10240
NSUBP = 2560
QBLK = 512
NQB = NP // QBLK
DBLK = 1024
NDB = NP // DBLK

_F32 = jnp.float32
_HI = lax.Precision.HIGHEST


def _mlp_sub_body(xs_ref, w_ref, b_ref, g_ref, be_ref, h_ref):
    y = jnp.dot(xs_ref[...], w_ref[...],
                preferred_element_type=_F32) + b_ref[...]
    mean = jnp.sum(y, axis=0, keepdims=True) / NSUB
    dev = y - mean
    var = jnp.sum(dev * dev, axis=0, keepdims=True) / NSUB
    hn = dev / jnp.sqrt(var + 1e-5)
    h_ref[...] = jnp.maximum(hn * g_ref[...] + be_ref[...], 0.0)


def _dense_body(xb_ref, w_ref, b_ref, g_ref, be_ref, y_ref, stats_ref, acc_ref):
    j = pl.program_id(0)
    y = jnp.dot(xb_ref[...], w_ref[...],
                preferred_element_type=_F32) + b_ref[...]
    y_ref[...] = y
    rid = lax.broadcasted_iota(jnp.int32, (DBLK, 1), 0)
    m = rid < (N - j * DBLK)
    ym = jnp.where(m, y, 0.0)

    @pl.when(j == 0)
    def _():
        acc_ref[...] = jnp.zeros_like(acc_ref)

    acc_ref[0:1, :] += jnp.sum(ym, axis=0, keepdims=True)
    acc_ref[1:2, :] += jnp.sum(ym * ym, axis=0, keepdims=True)

    @pl.when(j == NDB - 1)
    def _():
        mean = acc_ref[0:1, :] / N
        var = acc_ref[1:2, :] / N - mean * mean
        scale = g_ref[...] / jnp.sqrt(var + 1e-5)
        shift = be_ref[...] - mean * scale
        stats_ref[0:1, :] = scale
        stats_ref[1:2, :] = shift


def _knn_body(pq_ref, ps_ref, idx_ref, wn_ref):
    qx = pq_ref[:, 0:1]
    qy = pq_ref[:, 1:2]
    qz = pq_ref[:, 2:3]
    sx = ps_ref[0:1, :]
    sy = ps_ref[1:2, :]
    sz = ps_ref[2:3, :]
    def _bf(v):
        return v.astype(jnp.bfloat16).astype(_F32)
    dot = _bf(qx) * _bf(sx) + _bf(qy) * _bf(sy) + _bf(qz) * _bf(sz)
    d2 = ((qx * qx + qy * qy + qz * qz)
          + (sx * sx + sy * sy + sz * sz)
          - 2.0 * dot)
    ids = lax.broadcasted_iota(jnp.int32, (QBLK, NSUBP), 1)
    d = d2
    ams, ws = [], []
    for _ in range(3):
        mval = jnp.min(d, axis=1, keepdims=True)
        am = jnp.min(jnp.where(d == mval, ids, jnp.int32(2**30)),
                     axis=1, keepdims=True)
        ams.append(am)
        ws.append(1.0 / (jnp.maximum(mval, 0.0) + 1e-16))
        d = jnp.where(ids == am, jnp.float32(3e38), d)
    wsum = ws[0] + ws[1] + ws[2]
    zi = jnp.zeros((QBLK, 5), jnp.int32)
    idx_ref[...] = jnp.concatenate(ams + [zi], axis=1)
    wn_ref[...] = jnp.concatenate(
        [jnp.broadcast_to(w / wsum, (QBLK, 16)) for w in ws], axis=1)


_NC, _NS = 2, 16
_NW = _NC * _NS
_RPW = NP // _NW
_CH = 32
_NCH = _RPW // _CH


def _sc_interp_body(i0_hbm, i1_hbm, i2_hbm, wb_hbm, h_hbm, out_hbm,
                    i0_a, i1_a, i2_a, wb_a, r0_a, r1_a, r2_a,
                    i0_b, i1_b, i2_b, wb_b, r0_b, r1_b, r2_b,
                    out_v, sem_a, sem_b):
    wid = lax.axis_index("s") * _NC + lax.axis_index("c")
    base0 = wid * _RPW
    sets = [(i0_a, i1_a, i2_a, wb_a, r0_a, r1_a, r2_a, sem_a),
            (i0_b, i1_b, i2_b, wb_b, r0_b, r1_b, r2_b, sem_b)]

    def load(s, chunk):
        i0_v, i1_v, i2_v, wb_v, r0_v, r1_v, r2_v, sem = sets[s]
        sl = pl.ds(base0 + chunk * _CH, _CH)
        pltpu.sync_copy(i0_hbm.at[sl], i0_v)
        pltpu.sync_copy(i1_hbm.at[sl], i1_v)
        pltpu.sync_copy(i2_hbm.at[sl], i2_v)
        c0 = pltpu.async_copy(h_hbm.at[i0_v], r0_v, sem)
        c1 = pltpu.async_copy(h_hbm.at[i1_v], r1_v, sem)
        c2 = pltpu.async_copy(h_hbm.at[i2_v], r2_v, sem)
        pltpu.sync_copy(wb_hbm.at[sl], wb_v)
        return (c0, c1, c2)

    pend = {0: load(0, 0)}
    for chunk in range(_NCH):
        s = chunk & 1
        if chunk + 1 < _NCH:
            pend[1 - s] = load(1 - s, chunk + 1)
        for cp in pend[s]:
            cp.wait()
        _, _, _, wb_v, r0_v, r1_v, r2_v, _ = sets[s]

        def qbody(q, carry):
            w0 = wb_v[q, pl.ds(0, 16)]
            w1 = wb_v[q, pl.ds(16, 16)]
            w2 = wb_v[q, pl.ds(32, 16)]
            for c in range(COUT // 16):
                cs = pl.ds(c * 16, 16)
                out_v[q, cs] = (w0 * r0_v[q, cs] + w1 * r1_v[q, cs]
                                + w2 * r2_v[q, cs])
            return carry

        lax.fori_loop(0, _CH, qbody, 0, unroll=2)
        pltpu.sync_copy(out_v, out_hbm.at[pl.ds(base0 + chunk * _CH, _CH)])


def _sc_interp(idx0, idx1, idx2, wnb, h):
    mesh = plsc.VectorSubcoreMesh(core_axis_name="c", subcore_axis_name="s")
    dbuf = []
    for _ in range(2):
        dbuf += [
            pltpu.VMEM((_CH,), jnp.int32),
            pltpu.VMEM((_CH,), jnp.int32),
            pltpu.VMEM((_CH,), jnp.int32),
            pltpu.VMEM((_CH, 48), _F32),
            pltpu.VMEM((_CH, COUT), _F32),
            pltpu.VMEM((_CH, COUT), _F32),
            pltpu.VMEM((_CH, COUT), _F32),
        ]
    kfn = pl.kernel(
        _sc_interp_body,
        mesh=mesh,
        out_type=jax.ShapeDtypeStruct((NP, COUT), _F32),
        scratch_types=dbuf + [
            pltpu.VMEM((_CH, COUT), _F32),
            pltpu.SemaphoreType.DMA,
            pltpu.SemaphoreType.DMA,
        ],
    )
    return kfn(idx0, idx1, idx2, wnb, h)


def _combine_body(y_ref, stats_ref, interp_ref, out_ref):
    scale = stats_ref[0:1, :]
    shift = stats_ref[1:2, :]
    dn = jnp.maximum(y_ref[...] * scale + shift, 0.0)
    out_ref[...] = dn + interp_ref[...]


@jax.jit
def kernel(x, x_sub, pos, pos_sub, W_sub, b_sub, g_sub, be_sub, W, b, g, be):
    posq = jnp.zeros((NP, 8), _F32).at[:N, :3].set(pos)
    poss = jnp.full((8, NSUBP), 1e3, _F32).at[:3, :NSUB].set(pos_sub.T)

    h = pl.pallas_call(
        _mlp_sub_body,
        out_shape=jax.ShapeDtypeStruct((NSUB, COUT), _F32),
    )(x_sub, W_sub, b_sub, g_sub, be_sub)

    ybuf, stats = pl.pallas_call(
        _dense_body,
        grid=(NDB,),
        in_specs=[
            pl.BlockSpec((DBLK, COUT), lambda j: (j, 0)),
            pl.BlockSpec((COUT, COUT), lambda j: (0, 0)),
            pl.BlockSpec((1, COUT), lambda j: (0, 0)),
            pl.BlockSpec((1, COUT), lambda j: (0, 0)),
            pl.BlockSpec((1, COUT), lambda j: (0, 0)),
        ],
        out_specs=[
            pl.BlockSpec((DBLK, COUT), lambda j: (j, 0)),
            pl.BlockSpec((8, COUT), lambda j: (0, 0)),
        ],
        out_shape=[
            jax.ShapeDtypeStruct((N, COUT), _F32),
            jax.ShapeDtypeStruct((8, COUT), _F32),
        ],
        scratch_shapes=[pltpu.VMEM((8, COUT), _F32)],
    )(x, W, b.reshape(1, COUT), g.reshape(1, COUT), be.reshape(1, COUT))

    idxs, wns = pl.pallas_call(
        _knn_body,
        grid=(NQB,),
        in_specs=[
            pl.BlockSpec((QBLK, 8), lambda j: (j, 0)),
            pl.BlockSpec((8, NSUBP), lambda j: (0, 0)),
        ],
        out_specs=[
            pl.BlockSpec((QBLK, 8), lambda j: (j, 0)),
            pl.BlockSpec((QBLK, 48), lambda j: (j, 0)),
        ],
        out_shape=[
            jax.ShapeDtypeStruct((NP, 8), jnp.int32),
            jax.ShapeDtypeStruct((NP, 48), _F32),
        ],
    )(posq, poss)

    interp = _sc_interp(idxs[:, 0], idxs[:, 1], idxs[:, 2], wns, h)

    out = pl.pallas_call(
        _combine_body,
        grid=(NDB,),
        in_specs=[
            pl.BlockSpec((DBLK, COUT), lambda j: (j, 0)),
            pl.BlockSpec((8, COUT), lambda j: (0, 0)),
            pl.BlockSpec((DBLK, COUT), lambda j: (j, 0)),
        ],
        out_specs=pl.BlockSpec((DBLK, COUT), lambda j: (j, 0)),
        out_shape=jax.ShapeDtypeStruct((N, COUT), _F32),
    )(ybuf, stats, interp)
    return out


# speedup vs baseline: 3.3359x; 1.0232x over previous
import functools

import jax
import jax.numpy as jnp
from jax import lax
from jax.experimental import pallas as pl
from jax.experimental.pallas import tpu as pltpu
from jax.experimental.pallas import tpu_sc as plsc

N, NSUB, CIN, COUT = 10000, 2500, 512, 256
NP = 10240
NSUBP = 2560
QBLK = 512
NQB = NP // QBLK
DBLK = 1024
NDB = NP // DBLK

_F32 = jnp.float32
_HI = lax.Precision.HIGHEST


def _mlp_sub_body(xs_ref, w_ref, b_ref, g_ref, be_ref, h_ref):
    y = jnp.dot(xs_ref[...], w_ref[...],
                preferred_element_type=_F32) + b_ref[...]
    mean = jnp.sum(y, axis=0, keepdims=True) / NSUB
    dev = y - mean
    var = jnp.sum(dev * dev, axis=0, keepdims=True) / NSUB
    hn = dev / jnp.sqrt(var + 1e-5)
    h_ref[...] = jnp.maximum(hn * g_ref[...] + be_ref[...], 0.0)


def _dense_body(xb_ref, w_ref, b_ref, g_ref, be_ref, y_ref, stats_ref, acc_ref):
    j = pl.program_id(0)
    y = jnp.dot(xb_ref[...], w_ref[...],
                preferred_element_type=_F32) + b_ref[...]
    y_ref[...] = y
    rid = lax.broadcasted_iota(jnp.int32, (DBLK, 1), 0)
    m = rid < (N - j * DBLK)
    ym = jnp.where(m, y, 0.0)

    @pl.when(j == 0)
    def _():
        acc_ref[...] = jnp.zeros_like(acc_ref)

    acc_ref[0:1, :] += jnp.sum(ym, axis=0, keepdims=True)
    acc_ref[1:2, :] += jnp.sum(ym * ym, axis=0, keepdims=True)

    @pl.when(j == NDB - 1)
    def _():
        mean = acc_ref[0:1, :] / N
        var = acc_ref[1:2, :] / N - mean * mean
        scale = g_ref[...] / jnp.sqrt(var + 1e-5)
        shift = be_ref[...] - mean * scale
        stats_ref[0:1, :] = scale
        stats_ref[1:2, :] = shift


def _knn_body(pq_ref, ps_ref, idx_ref, wn_ref):
    qx = pq_ref[:, 0:1]
    qy = pq_ref[:, 1:2]
    qz = pq_ref[:, 2:3]
    sx = ps_ref[0:1, :]
    sy = ps_ref[1:2, :]
    sz = ps_ref[2:3, :]
    def _bf(v):
        return v.astype(jnp.bfloat16).astype(_F32)
    dot = _bf(qx) * _bf(sx) + _bf(qy) * _bf(sy) + _bf(qz) * _bf(sz)
    d2 = ((qx * qx + qy * qy + qz * qz)
          + (sx * sx + sy * sy + sz * sz)
          - 2.0 * dot)
    ids = lax.broadcasted_iota(jnp.int32, (QBLK, NSUBP), 1)
    d = d2
    ams, ws = [], []
    for _ in range(3):
        mval = jnp.min(d, axis=1, keepdims=True)
        am = jnp.min(jnp.where(d == mval, ids, jnp.int32(2**30)),
                     axis=1, keepdims=True)
        ams.append(am)
        ws.append(1.0 / (jnp.maximum(mval, 0.0) + 1e-16))
        d = jnp.where(ids == am, jnp.float32(3e38), d)
    wsum = ws[0] + ws[1] + ws[2]
    zi = jnp.zeros((QBLK, 5), jnp.int32)
    idx_ref[...] = jnp.concatenate(ams + [zi], axis=1)
    wn_ref[...] = jnp.concatenate(
        [jnp.broadcast_to(w / wsum, (QBLK, 16)) for w in ws], axis=1)


_NC, _NS = 2, 16
_NW = _NC * _NS
_RPW = NP // _NW
_CH = 32
_NCH = _RPW // _CH


def _sc_interp_body(nch, i0_hbm, i1_hbm, i2_hbm, wb_hbm, h_hbm, out_hbm,
                    i0_a, i1_a, i2_a, wb_a, r0_a, r1_a, r2_a,
                    i0_b, i1_b, i2_b, wb_b, r0_b, r1_b, r2_b,
                    out_v, sem_a, sem_b):
    _nch = nch
    wid = lax.axis_index("s") * _NC + lax.axis_index("c")
    base0 = wid * (_nch * _CH)
    sets = [(i0_a, i1_a, i2_a, wb_a, r0_a, r1_a, r2_a, sem_a),
            (i0_b, i1_b, i2_b, wb_b, r0_b, r1_b, r2_b, sem_b)]

    def load(s, chunk):
        i0_v, i1_v, i2_v, wb_v, r0_v, r1_v, r2_v, sem = sets[s]
        sl = pl.ds(base0 + chunk * _CH, _CH)
        pltpu.sync_copy(i0_hbm.at[sl], i0_v)
        pltpu.sync_copy(i1_hbm.at[sl], i1_v)
        pltpu.sync_copy(i2_hbm.at[sl], i2_v)
        c0 = pltpu.async_copy(h_hbm.at[i0_v], r0_v, sem)
        c1 = pltpu.async_copy(h_hbm.at[i1_v], r1_v, sem)
        c2 = pltpu.async_copy(h_hbm.at[i2_v], r2_v, sem)
        pltpu.sync_copy(wb_hbm.at[sl], wb_v)
        return (c0, c1, c2)

    pend = {0: load(0, 0)}
    for chunk in range(_nch):
        s = chunk & 1
        if chunk + 1 < _nch:
            pend[1 - s] = load(1 - s, chunk + 1)
        for cp in pend[s]:
            cp.wait()
        _, _, _, wb_v, r0_v, r1_v, r2_v, _ = sets[s]

        def qbody(q, carry):
            w0 = wb_v[q, pl.ds(0, 16)]
            w1 = wb_v[q, pl.ds(16, 16)]
            w2 = wb_v[q, pl.ds(32, 16)]
            for c in range(COUT // 16):
                cs = pl.ds(c * 16, 16)
                out_v[q, cs] = (w0 * r0_v[q, cs] + w1 * r1_v[q, cs]
                                + w2 * r2_v[q, cs])
            return carry

        lax.fori_loop(0, _CH, qbody, 0, unroll=2)
        pltpu.sync_copy(out_v, out_hbm.at[pl.ds(base0 + chunk * _CH, _CH)])


def _sc_interp(idx0, idx1, idx2, wnb, h):
    rows = idx0.shape[0]
    nch = rows // (_NW * _CH)
    mesh = plsc.VectorSubcoreMesh(core_axis_name="c", subcore_axis_name="s")
    dbuf = []
    for _ in range(2):
        dbuf += [
            pltpu.VMEM((_CH,), jnp.int32),
            pltpu.VMEM((_CH,), jnp.int32),
            pltpu.VMEM((_CH,), jnp.int32),
            pltpu.VMEM((_CH, 48), _F32),
            pltpu.VMEM((_CH, COUT), _F32),
            pltpu.VMEM((_CH, COUT), _F32),
            pltpu.VMEM((_CH, COUT), _F32),
        ]
    kfn = pl.kernel(
        functools.partial(_sc_interp_body, nch),
        mesh=mesh,
        out_type=jax.ShapeDtypeStruct((rows, COUT), _F32),
        scratch_types=dbuf + [
            pltpu.VMEM((_CH, COUT), _F32),
            pltpu.SemaphoreType.DMA,
            pltpu.SemaphoreType.DMA,
        ],
    )
    return kfn(idx0, idx1, idx2, wnb, h)


def _combine_body(y_ref, stats_ref, interp_ref, out_ref):
    scale = stats_ref[0:1, :]
    shift = stats_ref[1:2, :]
    dn = jnp.maximum(y_ref[...] * scale + shift, 0.0)
    out_ref[...] = dn + interp_ref[...]


@jax.jit
def kernel(x, x_sub, pos, pos_sub, W_sub, b_sub, g_sub, be_sub, W, b, g, be):
    posq = jnp.zeros((NP, 8), _F32).at[:N, :3].set(pos)
    poss = jnp.full((8, NSUBP), 1e3, _F32).at[:3, :NSUB].set(pos_sub.T)

    h = pl.pallas_call(
        _mlp_sub_body,
        out_shape=jax.ShapeDtypeStruct((NSUB, COUT), _F32),
    )(x_sub, W_sub, b_sub, g_sub, be_sub)

    ybuf, stats = pl.pallas_call(
        _dense_body,
        grid=(NDB,),
        in_specs=[
            pl.BlockSpec((DBLK, COUT), lambda j: (j, 0)),
            pl.BlockSpec((COUT, COUT), lambda j: (0, 0)),
            pl.BlockSpec((1, COUT), lambda j: (0, 0)),
            pl.BlockSpec((1, COUT), lambda j: (0, 0)),
            pl.BlockSpec((1, COUT), lambda j: (0, 0)),
        ],
        out_specs=[
            pl.BlockSpec((DBLK, COUT), lambda j: (j, 0)),
            pl.BlockSpec((8, COUT), lambda j: (0, 0)),
        ],
        out_shape=[
            jax.ShapeDtypeStruct((N, COUT), _F32),
            jax.ShapeDtypeStruct((8, COUT), _F32),
        ],
        scratch_shapes=[pltpu.VMEM((8, COUT), _F32)],
    )(x, W, b.reshape(1, COUT), g.reshape(1, COUT), be.reshape(1, COUT))

    HALF = NP // 2
    interps = []
    for half in range(2):
        idxs, wns = pl.pallas_call(
            _knn_body,
            grid=(HALF // QBLK,),
            in_specs=[
                pl.BlockSpec((QBLK, 8), lambda j: (j, 0)),
                pl.BlockSpec((8, NSUBP), lambda j: (0, 0)),
            ],
            out_specs=[
                pl.BlockSpec((QBLK, 8), lambda j: (j, 0)),
                pl.BlockSpec((QBLK, 48), lambda j: (j, 0)),
            ],
            out_shape=[
                jax.ShapeDtypeStruct((HALF, 8), jnp.int32),
                jax.ShapeDtypeStruct((HALF, 48), _F32),
            ],
        )(lax.dynamic_slice_in_dim(posq, half * HALF, HALF, 0), poss)
        interps.append(
            _sc_interp(idxs[:, 0], idxs[:, 1], idxs[:, 2], wns, h))

    outs = []
    row_splits = [(0, HALF), (HALF, N - HALF)]
    for (start, rows), interp in zip(row_splits, interps):
        nb = (rows + DBLK - 1) // DBLK
        outs.append(pl.pallas_call(
            _combine_body,
            grid=(nb,),
            in_specs=[
                pl.BlockSpec((DBLK, COUT), lambda j: (j, 0)),
                pl.BlockSpec((8, COUT), lambda j: (0, 0)),
                pl.BlockSpec((DBLK, COUT), lambda j: (j, 0)),
            ],
            out_specs=pl.BlockSpec((DBLK, COUT), lambda j: (j, 0)),
            out_shape=jax.ShapeDtypeStruct((rows, COUT), _F32),
        )(lax.dynamic_slice_in_dim(ybuf, start, rows, 0), stats, interp))
    return jnp.concatenate(outs, axis=0)


# speedup vs baseline: 3.7411x; 1.1215x over previous
import functools

import jax
import jax.numpy as jnp
from jax import lax
from jax.experimental import pallas as pl
from jax.experimental.pallas import tpu as pltpu
from jax.experimental.pallas import tpu_sc as plsc

N, NSUB, CIN, COUT = 10000, 2500, 512, 256
NP = 10240
NSUBP = 2560
QBLK = 512
NQB = NP // QBLK
DBLK = 1024
NDB = NP // DBLK

_F32 = jnp.float32
_HI = lax.Precision.HIGHEST


def _mlp_sub_body(xs_ref, w_ref, b_ref, g_ref, be_ref, h_ref):
    y = jnp.dot(xs_ref[...], w_ref[...],
                preferred_element_type=_F32) + b_ref[...]
    mean = jnp.sum(y, axis=0, keepdims=True) / NSUB
    dev = y - mean
    var = jnp.sum(dev * dev, axis=0, keepdims=True) / NSUB
    hn = dev / jnp.sqrt(var + 1e-5)
    h_ref[...] = jnp.maximum(hn * g_ref[...] + be_ref[...], 0.0)


def _dense_body(xb_ref, w_ref, b_ref, g_ref, be_ref, y_ref, stats_ref, acc_ref):
    j = pl.program_id(0)
    y = jnp.dot(xb_ref[...], w_ref[...],
                preferred_element_type=_F32) + b_ref[...]
    y_ref[...] = y
    rid = lax.broadcasted_iota(jnp.int32, (DBLK, 1), 0)
    m = rid < (N - j * DBLK)
    ym = jnp.where(m, y, 0.0)

    @pl.when(j == 0)
    def _():
        acc_ref[...] = jnp.zeros_like(acc_ref)

    acc_ref[0:1, :] += jnp.sum(ym, axis=0, keepdims=True)
    acc_ref[1:2, :] += jnp.sum(ym * ym, axis=0, keepdims=True)

    @pl.when(j == NDB - 1)
    def _():
        mean = acc_ref[0:1, :] / N
        var = acc_ref[1:2, :] / N - mean * mean
        scale = g_ref[...] / jnp.sqrt(var + 1e-5)
        shift = be_ref[...] - mean * scale
        stats_ref[0:1, :] = scale
        stats_ref[1:2, :] = shift


def _knn_body(pq_ref, ps_ref, idx_ref, wn_ref):
    qx = pq_ref[:, 0:1]
    qy = pq_ref[:, 1:2]
    qz = pq_ref[:, 2:3]
    sx = ps_ref[0:1, :]
    sy = ps_ref[1:2, :]
    sz = ps_ref[2:3, :]
    def _bf(v):
        return v.astype(jnp.bfloat16).astype(_F32)
    dot = _bf(qx) * _bf(sx) + _bf(qy) * _bf(sy) + _bf(qz) * _bf(sz)
    d2 = ((qx * qx + qy * qy + qz * qz)
          + (sx * sx + sy * sy + sz * sz)
          - 2.0 * dot)
    ids = lax.broadcasted_iota(jnp.int32, (QBLK, NSUBP), 1)
    d = d2
    ams, ws = [], []
    for _ in range(3):
        mval = jnp.min(d, axis=1, keepdims=True)
        am = jnp.min(jnp.where(d == mval, ids, jnp.int32(2**30)),
                     axis=1, keepdims=True)
        ams.append(am)
        ws.append(1.0 / (jnp.maximum(mval, 0.0) + 1e-16))
        d = jnp.where(ids == am, jnp.float32(3e38), d)
    wsum = ws[0] + ws[1] + ws[2]
    zi = jnp.zeros((5, QBLK), jnp.int32)
    idx_ref[...] = jnp.concatenate(
        [jnp.swapaxes(am, 0, 1) for am in ams] + [zi], axis=0)
    wn_ref[...] = jnp.concatenate(
        [jnp.broadcast_to(w / wsum, (QBLK, 16)) for w in ws], axis=1)


_NC, _NS = 2, 16
_NW = _NC * _NS
_RPW = NP // _NW
_CH = 32
_NCH = _RPW // _CH


def _sc_interp_body(nch, it_hbm, wb_hbm, h_hbm, out_hbm,
                    i0_a, i1_a, i2_a, wb_a, r0_a, r1_a, r2_a,
                    i0_b, i1_b, i2_b, wb_b, r0_b, r1_b, r2_b,
                    out_v, sem_a, sem_b):
    _nch = nch
    wid = lax.axis_index("s") * _NC + lax.axis_index("c")
    base0 = wid * (_nch * _CH)
    sets = [(i0_a, i1_a, i2_a, wb_a, r0_a, r1_a, r2_a, sem_a),
            (i0_b, i1_b, i2_b, wb_b, r0_b, r1_b, r2_b, sem_b)]

    def load(s, chunk):
        i0_v, i1_v, i2_v, wb_v, r0_v, r1_v, r2_v, sem = sets[s]
        sl = pl.ds(base0 + chunk * _CH, _CH)
        pltpu.sync_copy(it_hbm.at[0, sl], i0_v)
        pltpu.sync_copy(it_hbm.at[1, sl], i1_v)
        pltpu.sync_copy(it_hbm.at[2, sl], i2_v)
        c0 = pltpu.async_copy(h_hbm.at[i0_v], r0_v, sem)
        c1 = pltpu.async_copy(h_hbm.at[i1_v], r1_v, sem)
        c2 = pltpu.async_copy(h_hbm.at[i2_v], r2_v, sem)
        pltpu.sync_copy(wb_hbm.at[sl], wb_v)
        return (c0, c1, c2)

    pend = {0: load(0, 0)}
    for chunk in range(_nch):
        s = chunk & 1
        if chunk + 1 < _nch:
            pend[1 - s] = load(1 - s, chunk + 1)
        for cp in pend[s]:
            cp.wait()
        _, _, _, wb_v, r0_v, r1_v, r2_v, _ = sets[s]

        def qbody(q, carry):
            w0 = wb_v[q, pl.ds(0, 16)]
            w1 = wb_v[q, pl.ds(16, 16)]
            w2 = wb_v[q, pl.ds(32, 16)]
            for c in range(COUT // 16):
                cs = pl.ds(c * 16, 16)
                out_v[q, cs] = (w0 * r0_v[q, cs] + w1 * r1_v[q, cs]
                                + w2 * r2_v[q, cs])
            return carry

        lax.fori_loop(0, _CH, qbody, 0, unroll=2)
        pltpu.sync_copy(out_v, out_hbm.at[pl.ds(base0 + chunk * _CH, _CH)])


def _sc_interp(idx_t, wnb, h):
    rows = idx_t.shape[1]
    nch = rows // (_NW * _CH)
    mesh = plsc.VectorSubcoreMesh(core_axis_name="c", subcore_axis_name="s")
    dbuf = []
    for _ in range(2):
        dbuf += [
            pltpu.VMEM((_CH,), jnp.int32),
            pltpu.VMEM((_CH,), jnp.int32),
            pltpu.VMEM((_CH,), jnp.int32),
            pltpu.VMEM((_CH, 48), _F32),
            pltpu.VMEM((_CH, COUT), _F32),
            pltpu.VMEM((_CH, COUT), _F32),
            pltpu.VMEM((_CH, COUT), _F32),
        ]
    kfn = pl.kernel(
        functools.partial(_sc_interp_body, nch),
        mesh=mesh,
        out_type=jax.ShapeDtypeStruct((rows, COUT), _F32),
        scratch_types=dbuf + [
            pltpu.VMEM((_CH, COUT), _F32),
            pltpu.SemaphoreType.DMA,
            pltpu.SemaphoreType.DMA,
        ],
    )
    return kfn(idx_t, wnb, h)


def _combine_body(y_ref, stats_ref, ia_ref, ib_ref, out_ref):
    j = pl.program_id(0)
    scale = stats_ref[0:1, :]
    shift = stats_ref[1:2, :]
    dn = jnp.maximum(y_ref[...] * scale + shift, 0.0)
    interp = jnp.where(j < NDB // 2, ia_ref[...], ib_ref[...])
    out_ref[...] = dn + interp


@jax.jit
def kernel(x, x_sub, pos, pos_sub, W_sub, b_sub, g_sub, be_sub, W, b, g, be):
    HALF = NP // 2
    posq_halves = [
        jnp.zeros((HALF, 8), _F32).at[:, :3].set(pos[:HALF]),
        jnp.zeros((HALF, 8), _F32).at[:N - HALF, :3].set(pos[HALF:]),
    ]
    poss = jnp.full((8, NSUBP), 1e3, _F32).at[:3, :NSUB].set(pos_sub.T)

    h = pl.pallas_call(
        _mlp_sub_body,
        out_shape=jax.ShapeDtypeStruct((NSUB, COUT), _F32),
    )(x_sub, W_sub, b_sub, g_sub, be_sub)

    ybuf, stats = pl.pallas_call(
        _dense_body,
        grid=(NDB,),
        in_specs=[
            pl.BlockSpec((DBLK, COUT), lambda j: (j, 0)),
            pl.BlockSpec((COUT, COUT), lambda j: (0, 0)),
            pl.BlockSpec((1, COUT), lambda j: (0, 0)),
            pl.BlockSpec((1, COUT), lambda j: (0, 0)),
            pl.BlockSpec((1, COUT), lambda j: (0, 0)),
        ],
        out_specs=[
            pl.BlockSpec((DBLK, COUT), lambda j: (j, 0)),
            pl.BlockSpec((8, COUT), lambda j: (0, 0)),
        ],
        out_shape=[
            jax.ShapeDtypeStruct((N, COUT), _F32),
            jax.ShapeDtypeStruct((8, COUT), _F32),
        ],
        scratch_shapes=[pltpu.VMEM((8, COUT), _F32)],
    )(x, W, b.reshape(1, COUT), g.reshape(1, COUT), be.reshape(1, COUT))

    interps = []
    for half in range(2):
        idx_t, wns = pl.pallas_call(
            _knn_body,
            grid=(HALF // QBLK,),
            in_specs=[
                pl.BlockSpec((QBLK, 8), lambda j: (j, 0)),
                pl.BlockSpec((8, NSUBP), lambda j: (0, 0)),
            ],
            out_specs=[
                pl.BlockSpec((8, QBLK), lambda j: (0, j)),
                pl.BlockSpec((QBLK, 48), lambda j: (j, 0)),
            ],
            out_shape=[
                jax.ShapeDtypeStruct((8, HALF), jnp.int32),
                jax.ShapeDtypeStruct((HALF, 48), _F32),
            ],
        )(posq_halves[half], poss)
        interps.append(_sc_interp(idx_t, wns, h))

    nhb = NDB // 2
    out = pl.pallas_call(
        _combine_body,
        grid=(NDB,),
        in_specs=[
            pl.BlockSpec((DBLK, COUT), lambda j: (j, 0)),
            pl.BlockSpec((8, COUT), lambda j: (0, 0)),
            pl.BlockSpec((DBLK, COUT), lambda j: (jnp.minimum(j, nhb - 1), 0)),
            pl.BlockSpec((DBLK, COUT), lambda j: (jnp.maximum(j - nhb, 0), 0)),
        ],
        out_specs=pl.BlockSpec((DBLK, COUT), lambda j: (j, 0)),
        out_shape=jax.ShapeDtypeStruct((N, COUT), _F32),
    )(ybuf, stats, interps[0], interps[1])
    return out


# speedup vs baseline: 4.1240x; 1.1023x over previous
import functools

import jax
import jax.numpy as jnp
from jax import lax
from jax.experimental import pallas as pl
from jax.experimental.pallas import tpu as pltpu
from jax.experimental.pallas import tpu_sc as plsc

N, NSUB, CIN, COUT = 10000, 2500, 512, 256
NP = 10240
NSUBP = 2560
QBLK = 512
NQB = NP // QBLK
DBLK = 1024
NDB = NP // DBLK

_F32 = jnp.float32
_HI = lax.Precision.HIGHEST


def _mlp_sub_body(xs_ref, w_ref, b_ref, g_ref, be_ref, h_ref):
    y = jnp.dot(xs_ref[...], w_ref[...],
                preferred_element_type=_F32) + b_ref[...]
    mean = jnp.sum(y, axis=0, keepdims=True) / NSUB
    dev = y - mean
    var = jnp.sum(dev * dev, axis=0, keepdims=True) / NSUB
    hn = dev / jnp.sqrt(var + 1e-5)
    h_ref[...] = jnp.maximum(hn * g_ref[...] + be_ref[...], 0.0)


def _dense_body(xb_ref, w_ref, b_ref, g_ref, be_ref, y_ref, stats_ref, acc_ref):
    j = pl.program_id(0)
    y = jnp.dot(xb_ref[...], w_ref[...],
                preferred_element_type=_F32) + b_ref[...]
    y_ref[...] = y
    rid = lax.broadcasted_iota(jnp.int32, (DBLK, 1), 0)
    m = rid < (N - j * DBLK)
    ym = jnp.where(m, y, 0.0)

    @pl.when(j == 0)
    def _():
        acc_ref[...] = jnp.zeros_like(acc_ref)

    acc_ref[0:1, :] += jnp.sum(ym, axis=0, keepdims=True)
    acc_ref[1:2, :] += jnp.sum(ym * ym, axis=0, keepdims=True)

    @pl.when(j == NDB - 1)
    def _():
        mean = acc_ref[0:1, :] / N
        var = acc_ref[1:2, :] / N - mean * mean
        scale = g_ref[...] / jnp.sqrt(var + 1e-5)
        shift = be_ref[...] - mean * scale
        stats_ref[0:1, :] = scale
        stats_ref[1:2, :] = shift


_KC = 128
_NKC = NSUBP // _KC
_BIGF = 3e38
_BIGI = 2**30


def _knn_body(pq_ref, ps_ref, idx_ref, wn_ref):
    qx = pq_ref[:, 0:1]
    qy = pq_ref[:, 1:2]
    qz = pq_ref[:, 2:3]
    sx = ps_ref[0:1, :]
    sy = ps_ref[1:2, :]
    sz = ps_ref[2:3, :]
    dot = jnp.dot(pq_ref[...], ps_ref[...],
                  preferred_element_type=_F32)
    d2 = ((qx * qx + qy * qy + qz * qz)
          + (sx * sx + sy * sy + sz * sz)
          - 2.0 * dot)
    ids = lax.broadcasted_iota(jnp.int32, (QBLK, NSUBP), 1)
    d = d2
    ams, ws = [], []
    for _ in range(3):
        mval = jnp.min(d, axis=1, keepdims=True)
        am = jnp.min(jnp.where(d == mval, ids, _BIGI),
                     axis=1, keepdims=True)
        ams.append(am)
        ws.append(1.0 / (jnp.maximum(mval, 0.0) + 1e-16))
        d = jnp.where(ids == am, _BIGF, d)
    wsum = ws[0] + ws[1] + ws[2]
    zi = jnp.zeros((5, QBLK), jnp.int32)
    idx_ref[...] = jnp.concatenate(
        [jnp.swapaxes(am, 0, 1) for am in ams] + [zi], axis=0)
    wn_ref[...] = jnp.concatenate(
        [jnp.broadcast_to(w / wsum, (QBLK, 16)) for w in ws], axis=1)


_NC, _NS = 2, 16
_NW = _NC * _NS
_RPW = NP // _NW
_CH = 32
_NCH = _RPW // _CH


def _sc_interp_body(nch, it_hbm, wb_hbm, h_hbm, out_hbm,
                    i0_a, i1_a, i2_a, wb_a, r0_a, r1_a, r2_a,
                    i0_b, i1_b, i2_b, wb_b, r0_b, r1_b, r2_b,
                    out_v, sem_a, sem_b):
    _nch = nch
    wid = lax.axis_index("s") * _NC + lax.axis_index("c")
    base0 = wid * (_nch * _CH)
    sets = [(i0_a, i1_a, i2_a, wb_a, r0_a, r1_a, r2_a, sem_a),
            (i0_b, i1_b, i2_b, wb_b, r0_b, r1_b, r2_b, sem_b)]

    def load(s, chunk):
        i0_v, i1_v, i2_v, wb_v, r0_v, r1_v, r2_v, sem = sets[s]
        sl = pl.ds(base0 + chunk * _CH, _CH)
        pltpu.sync_copy(it_hbm.at[0, sl], i0_v)
        pltpu.sync_copy(it_hbm.at[1, sl], i1_v)
        pltpu.sync_copy(it_hbm.at[2, sl], i2_v)
        c0 = pltpu.async_copy(h_hbm.at[i0_v], r0_v, sem)
        c1 = pltpu.async_copy(h_hbm.at[i1_v], r1_v, sem)
        c2 = pltpu.async_copy(h_hbm.at[i2_v], r2_v, sem)
        pltpu.sync_copy(wb_hbm.at[sl], wb_v)
        return (c0, c1, c2)

    pend = {0: load(0, 0)}
    for chunk in range(_nch):
        s = chunk & 1
        if chunk + 1 < _nch:
            pend[1 - s] = load(1 - s, chunk + 1)
        for cp in pend[s]:
            cp.wait()
        _, _, _, wb_v, r0_v, r1_v, r2_v, _ = sets[s]

        def qbody(q, carry):
            w0 = wb_v[q, pl.ds(0, 16)]
            w1 = wb_v[q, pl.ds(16, 16)]
            w2 = wb_v[q, pl.ds(32, 16)]
            for c in range(COUT // 16):
                cs = pl.ds(c * 16, 16)
                out_v[q, cs] = (w0 * r0_v[q, cs] + w1 * r1_v[q, cs]
                                + w2 * r2_v[q, cs])
            return carry

        lax.fori_loop(0, _CH, qbody, 0, unroll=2)
        pltpu.sync_copy(out_v, out_hbm.at[pl.ds(base0 + chunk * _CH, _CH)])


def _sc_interp(idx_t, wnb, h):
    rows = idx_t.shape[1]
    nch = rows // (_NW * _CH)
    mesh = plsc.VectorSubcoreMesh(core_axis_name="c", subcore_axis_name="s")
    dbuf = []
    for _ in range(2):
        dbuf += [
            pltpu.VMEM((_CH,), jnp.int32),
            pltpu.VMEM((_CH,), jnp.int32),
            pltpu.VMEM((_CH,), jnp.int32),
            pltpu.VMEM((_CH, 48), _F32),
            pltpu.VMEM((_CH, COUT), _F32),
            pltpu.VMEM((_CH, COUT), _F32),
            pltpu.VMEM((_CH, COUT), _F32),
        ]
    kfn = pl.kernel(
        functools.partial(_sc_interp_body, nch),
        mesh=mesh,
        out_type=jax.ShapeDtypeStruct((rows, COUT), _F32),
        scratch_types=dbuf + [
            pltpu.VMEM((_CH, COUT), _F32),
            pltpu.SemaphoreType.DMA,
            pltpu.SemaphoreType.DMA,
        ],
    )
    return kfn(idx_t, wnb, h)


def _combine_body(y_ref, stats_ref, ia_ref, ib_ref, out_ref):
    j = pl.program_id(0)
    scale = stats_ref[0:1, :]
    shift = stats_ref[1:2, :]
    dn = jnp.maximum(y_ref[...] * scale + shift, 0.0)
    interp = jnp.where(j < NDB // 2, ia_ref[...], ib_ref[...])
    out_ref[...] = dn + interp


@jax.jit
def kernel(x, x_sub, pos, pos_sub, W_sub, b_sub, g_sub, be_sub, W, b, g, be):
    HALF = NP // 2
    posq_halves = [
        jnp.zeros((HALF, 8), _F32).at[:, :3].set(pos[:HALF]),
        jnp.zeros((HALF, 8), _F32).at[:N - HALF, :3].set(pos[HALF:]),
    ]
    poss = jnp.full((8, NSUBP), 1e3, _F32).at[:3, :NSUB].set(pos_sub.T)

    h = pl.pallas_call(
        _mlp_sub_body,
        out_shape=jax.ShapeDtypeStruct((NSUB, COUT), _F32),
    )(x_sub, W_sub, b_sub, g_sub, be_sub)

    ybuf, stats = pl.pallas_call(
        _dense_body,
        grid=(NDB,),
        in_specs=[
            pl.BlockSpec((DBLK, COUT), lambda j: (j, 0)),
            pl.BlockSpec((COUT, COUT), lambda j: (0, 0)),
            pl.BlockSpec((1, COUT), lambda j: (0, 0)),
            pl.BlockSpec((1, COUT), lambda j: (0, 0)),
            pl.BlockSpec((1, COUT), lambda j: (0, 0)),
        ],
        out_specs=[
            pl.BlockSpec((DBLK, COUT), lambda j: (j, 0)),
            pl.BlockSpec((8, COUT), lambda j: (0, 0)),
        ],
        out_shape=[
            jax.ShapeDtypeStruct((N, COUT), _F32),
            jax.ShapeDtypeStruct((8, COUT), _F32),
        ],
        scratch_shapes=[pltpu.VMEM((8, COUT), _F32)],
    )(x, W, b.reshape(1, COUT), g.reshape(1, COUT), be.reshape(1, COUT))

    interps = []
    for half in range(2):
        idx_t, wns = pl.pallas_call(
            _knn_body,
            grid=(HALF // QBLK,),
            in_specs=[
                pl.BlockSpec((QBLK, 8), lambda j: (j, 0)),
                pl.BlockSpec((8, NSUBP), lambda j: (0, 0)),
            ],
            out_specs=[
                pl.BlockSpec((8, QBLK), lambda j: (0, j)),
                pl.BlockSpec((QBLK, 48), lambda j: (j, 0)),
            ],
            out_shape=[
                jax.ShapeDtypeStruct((8, HALF), jnp.int32),
                jax.ShapeDtypeStruct((HALF, 48), _F32),
            ],
        )(posq_halves[half], poss)
        interps.append(_sc_interp(idx_t, wns, h))

    nhb = NDB // 2
    out = pl.pallas_call(
        _combine_body,
        grid=(NDB,),
        in_specs=[
            pl.BlockSpec((DBLK, COUT), lambda j: (j, 0)),
            pl.BlockSpec((8, COUT), lambda j: (0, 0)),
            pl.BlockSpec((DBLK, COUT), lambda j: (jnp.minimum(j, nhb - 1), 0)),
            pl.BlockSpec((DBLK, COUT), lambda j: (jnp.maximum(j - nhb, 0), 0)),
        ],
        out_specs=pl.BlockSpec((DBLK, COUT), lambda j: (j, 0)),
        out_shape=jax.ShapeDtypeStruct((N, COUT), _F32),
    )(ybuf, stats, interps[0], interps[1])
    return out


# speedup vs baseline: 4.1619x; 1.0092x over previous
import functools

import jax
import jax.numpy as jnp
from jax import lax
from jax.experimental import pallas as pl
from jax.experimental.pallas import tpu as pltpu
from jax.experimental.pallas import tpu_sc as plsc

N, NSUB, CIN, COUT = 10000, 2500, 512, 256
NP = 10240
NSUBP = 2560
QBLK = 1024
NQB = NP // QBLK
DBLK = 1024
NDB = NP // DBLK

_F32 = jnp.float32
_HI = lax.Precision.HIGHEST


def _mlp_sub_body(xs_ref, w_ref, b_ref, g_ref, be_ref, h_ref):
    y = jnp.dot(xs_ref[...], w_ref[...],
                preferred_element_type=_F32) + b_ref[...]
    mean = jnp.sum(y, axis=0, keepdims=True) / NSUB
    dev = y - mean
    var = jnp.sum(dev * dev, axis=0, keepdims=True) / NSUB
    hn = dev / jnp.sqrt(var + 1e-5)
    h_ref[...] = jnp.maximum(hn * g_ref[...] + be_ref[...], 0.0)


def _dense_body(xb_ref, w_ref, b_ref, g_ref, be_ref, y_ref, stats_ref, acc_ref):
    j = pl.program_id(0)
    y = jnp.dot(xb_ref[...], w_ref[...],
                preferred_element_type=_F32) + b_ref[...]
    y_ref[...] = y
    rid = lax.broadcasted_iota(jnp.int32, (DBLK, 1), 0)
    m = rid < (N - j * DBLK)
    ym = jnp.where(m, y, 0.0)

    @pl.when(j == 0)
    def _():
        acc_ref[...] = jnp.zeros_like(acc_ref)

    acc_ref[0:1, :] += jnp.sum(ym, axis=0, keepdims=True)
    acc_ref[1:2, :] += jnp.sum(ym * ym, axis=0, keepdims=True)

    @pl.when(j == NDB - 1)
    def _():
        mean = acc_ref[0:1, :] / N
        var = acc_ref[1:2, :] / N - mean * mean
        scale = g_ref[...] / jnp.sqrt(var + 1e-5)
        shift = be_ref[...] - mean * scale
        stats_ref[0:1, :] = scale
        stats_ref[1:2, :] = shift


_KC = 128
_NKC = NSUBP // _KC
_BIGF = 3e38
_BIGI = 2**30


def _knn_body(pq_ref, ps_ref, idx_ref, wn_ref):
    qx = pq_ref[:, 0:1]
    qy = pq_ref[:, 1:2]
    qz = pq_ref[:, 2:3]
    sx = ps_ref[0:1, :]
    sy = ps_ref[1:2, :]
    sz = ps_ref[2:3, :]
    dot = jnp.dot(pq_ref[:, 0:3], ps_ref[0:3, :],
                  preferred_element_type=_F32)
    d2 = ((qx * qx + qy * qy + qz * qz)
          + (sx * sx + sy * sy + sz * sz)
          - 2.0 * dot)
    ids = lax.broadcasted_iota(jnp.int32, (QBLK, NSUBP), 1)
    d = d2
    ams, ws = [], []
    for _ in range(3):
        mval = jnp.min(d, axis=1, keepdims=True)
        am = jnp.min(jnp.where(d == mval, ids, _BIGI),
                     axis=1, keepdims=True)
        ams.append(am)
        ws.append(1.0 / (jnp.maximum(mval, 0.0) + 1e-16))
        d = jnp.where(ids == am, _BIGF, d)
    wsum = ws[0] + ws[1] + ws[2]
    zi = jnp.zeros((5, QBLK), jnp.int32)
    idx_ref[...] = jnp.concatenate(
        [jnp.swapaxes(am, 0, 1) for am in ams] + [zi], axis=0)
    wn_ref[...] = jnp.concatenate(
        [jnp.broadcast_to(w / wsum, (QBLK, 16)) for w in ws], axis=1)


_NC, _NS = 2, 16
_NW = _NC * _NS
_RPW = NP // _NW
_CH = 32
_NCH = _RPW // _CH


def _sc_interp_body(nch, it_hbm, wb_hbm, h_hbm, out_hbm,
                    i0_a, i1_a, i2_a, wb_a, r0_a, r1_a, r2_a,
                    i0_b, i1_b, i2_b, wb_b, r0_b, r1_b, r2_b,
                    out_v, sem_a, sem_b):
    _nch = nch
    wid = lax.axis_index("s") * _NC + lax.axis_index("c")
    base0 = wid * (_nch * _CH)
    sets = [(i0_a, i1_a, i2_a, wb_a, r0_a, r1_a, r2_a, sem_a),
            (i0_b, i1_b, i2_b, wb_b, r0_b, r1_b, r2_b, sem_b)]

    def load(s, chunk):
        i0_v, i1_v, i2_v, wb_v, r0_v, r1_v, r2_v, sem = sets[s]
        sl = pl.ds(base0 + chunk * _CH, _CH)
        pltpu.sync_copy(it_hbm.at[0, sl], i0_v)
        pltpu.sync_copy(it_hbm.at[1, sl], i1_v)
        pltpu.sync_copy(it_hbm.at[2, sl], i2_v)
        c0 = pltpu.async_copy(h_hbm.at[i0_v], r0_v, sem)
        c1 = pltpu.async_copy(h_hbm.at[i1_v], r1_v, sem)
        c2 = pltpu.async_copy(h_hbm.at[i2_v], r2_v, sem)
        pltpu.sync_copy(wb_hbm.at[sl], wb_v)
        return (c0, c1, c2)

    pend = {0: load(0, 0)}
    for chunk in range(_nch):
        s = chunk & 1
        if chunk + 1 < _nch:
            pend[1 - s] = load(1 - s, chunk + 1)
        for cp in pend[s]:
            cp.wait()
        _, _, _, wb_v, r0_v, r1_v, r2_v, _ = sets[s]

        def qbody(q, carry):
            w0 = wb_v[q, pl.ds(0, 16)]
            w1 = wb_v[q, pl.ds(16, 16)]
            w2 = wb_v[q, pl.ds(32, 16)]
            for c in range(COUT // 16):
                cs = pl.ds(c * 16, 16)
                out_v[q, cs] = (w0 * r0_v[q, cs] + w1 * r1_v[q, cs]
                                + w2 * r2_v[q, cs])
            return carry

        lax.fori_loop(0, _CH, qbody, 0, unroll=2)
        pltpu.sync_copy(out_v, out_hbm.at[pl.ds(base0 + chunk * _CH, _CH)])


def _sc_interp(idx_t, wnb, h):
    rows = idx_t.shape[1]
    nch = rows // (_NW * _CH)
    mesh = plsc.VectorSubcoreMesh(core_axis_name="c", subcore_axis_name="s")
    dbuf = []
    for _ in range(2):
        dbuf += [
            pltpu.VMEM((_CH,), jnp.int32),
            pltpu.VMEM((_CH,), jnp.int32),
            pltpu.VMEM((_CH,), jnp.int32),
            pltpu.VMEM((_CH, 48), _F32),
            pltpu.VMEM((_CH, COUT), _F32),
            pltpu.VMEM((_CH, COUT), _F32),
            pltpu.VMEM((_CH, COUT), _F32),
        ]
    kfn = pl.kernel(
        functools.partial(_sc_interp_body, nch),
        mesh=mesh,
        out_type=jax.ShapeDtypeStruct((rows, COUT), _F32),
        scratch_types=dbuf + [
            pltpu.VMEM((_CH, COUT), _F32),
            pltpu.SemaphoreType.DMA,
            pltpu.SemaphoreType.DMA,
        ],
    )
    return kfn(idx_t, wnb, h)


def _combine_body(y_ref, stats_ref, ia_ref, ib_ref, out_ref):
    j = pl.program_id(0)
    scale = stats_ref[0:1, :]
    shift = stats_ref[1:2, :]
    dn = jnp.maximum(y_ref[...] * scale + shift, 0.0)
    interp = jnp.where(j < NDB // 2, ia_ref[...], ib_ref[...])
    out_ref[...] = dn + interp


@jax.jit
def kernel(x, x_sub, pos, pos_sub, W_sub, b_sub, g_sub, be_sub, W, b, g, be):
    HALF = NP // 2
    posq_halves = [
        jnp.zeros((HALF, 8), _F32).at[:, :3].set(pos[:HALF]),
        jnp.zeros((HALF, 8), _F32).at[:N - HALF, :3].set(pos[HALF:]),
    ]
    poss = jnp.full((8, NSUBP), 1e3, _F32).at[:3, :NSUB].set(pos_sub.T)

    h = pl.pallas_call(
        _mlp_sub_body,
        out_shape=jax.ShapeDtypeStruct((NSUB, COUT), _F32),
    )(x_sub, W_sub, b_sub, g_sub, be_sub)

    ybuf, stats = pl.pallas_call(
        _dense_body,
        grid=(NDB,),
        in_specs=[
            pl.BlockSpec((DBLK, COUT), lambda j: (j, 0)),
            pl.BlockSpec((COUT, COUT), lambda j: (0, 0)),
            pl.BlockSpec((1, COUT), lambda j: (0, 0)),
            pl.BlockSpec((1, COUT), lambda j: (0, 0)),
            pl.BlockSpec((1, COUT), lambda j: (0, 0)),
        ],
        out_specs=[
            pl.BlockSpec((DBLK, COUT), lambda j: (j, 0)),
            pl.BlockSpec((8, COUT), lambda j: (0, 0)),
        ],
        out_shape=[
            jax.ShapeDtypeStruct((N, COUT), _F32),
            jax.ShapeDtypeStruct((8, COUT), _F32),
        ],
        scratch_shapes=[pltpu.VMEM((8, COUT), _F32)],
    )(x, W, b.reshape(1, COUT), g.reshape(1, COUT), be.reshape(1, COUT))

    interps = []
    for half in range(2):
        idx_t, wns = pl.pallas_call(
            _knn_body,
            grid=(HALF // QBLK,),
            in_specs=[
                pl.BlockSpec((QBLK, 8), lambda j: (j, 0)),
                pl.BlockSpec((8, NSUBP), lambda j: (0, 0)),
            ],
            out_specs=[
                pl.BlockSpec((8, QBLK), lambda j: (0, j)),
                pl.BlockSpec((QBLK, 48), lambda j: (j, 0)),
            ],
            out_shape=[
                jax.ShapeDtypeStruct((8, HALF), jnp.int32),
                jax.ShapeDtypeStruct((HALF, 48), _F32),
            ],
        )(posq_halves[half], poss)
        interps.append(_sc_interp(idx_t, wns, h))

    nhb = NDB // 2
    out = pl.pallas_call(
        _combine_body,
        grid=(NDB,),
        in_specs=[
            pl.BlockSpec((DBLK, COUT), lambda j: (j, 0)),
            pl.BlockSpec((8, COUT), lambda j: (0, 0)),
            pl.BlockSpec((DBLK, COUT), lambda j: (jnp.minimum(j, nhb - 1), 0)),
            pl.BlockSpec((DBLK, COUT), lambda j: (jnp.maximum(j - nhb, 0), 0)),
        ],
        out_specs=pl.BlockSpec((DBLK, COUT), lambda j: (j, 0)),
        out_shape=jax.ShapeDtypeStruct((N, COUT), _F32),
    )(ybuf, stats, interps[0], interps[1])
    return out
